# baseline (speedup 1.0000x reference)
import jax
import jax.numpy as jnp
from jax import lax
from jax.experimental import pallas as pl
from jax.experimental.pallas import tpu as pltpu
from jax.experimental.pallas import tpu_sc as plsc

D = 768
B = 16384
NC = 2
NS = 16
NW = NC * NS
PER_W = B // NW
C = 16
NCHUNK = PER_W // C
NBUF = 3
LANES = 16
DCH = D // LANES


def _sqrt16(x):
    i = plsc.bitcast(x, jnp.int32)
    y = plsc.bitcast((i >> 1) + 0x1FBD1DF6, jnp.float32)
    for _ in range(3):
        y = 0.5 * (y + x / y)
    return y


def _body(ee, rel_t, ps_h, out_h,
          hcol, tcol, hidx, ridx, hbuf, rbuf, osum, sems):
    wid = lax.axis_index("s") * NC + lax.axis_index("c")

    pltpu.sync_copy(ps_h.at[0, wid], hcol)
    pltpu.sync_copy(ps_h.at[2, wid], tcol)
    pltpu.sync_copy(ps_h.at[1, wid], ridx)

    def mk(g, carry):
        hidx[g, pl.ds(0, C)] = hcol[g, :]
        hidx[g, pl.ds(C, C)] = tcol[g, :]
        return carry

    lax.fori_loop(0, NCHUNK, mk, 0)

    def start(g, b):
        pltpu.async_copy(ee.at[hidx.at[g]], hbuf.at[b], sems.at[b])
        pltpu.async_copy(rel_t.at[ridx[g, :]], rbuf.at[b], sems.at[b])

    def wait(g, b):
        pltpu.make_async_copy(ee.at[hidx.at[g]], hbuf.at[b],
                              sems.at[b]).wait()
        pltpu.make_async_copy(rel_t.at[ridx[g, :]], rbuf.at[b],
                              sems.at[b]).wait()

    for b in range(NBUF):
        start(b, b)

    lanes = lax.iota(jnp.int32, LANES)

    def step(g, carry):
        b = lax.rem(g, NBUF)
        wait(g, b)

        def row(r, res):
            def dstep(j, acc):
                sl = pl.ds(j * LANES, LANES)
                v = hbuf[b, r, sl] + rbuf[b, r, sl] - hbuf[b, C + r, sl]
                return acc + v * v
            acc = lax.fori_loop(0, DCH, dstep,
                                jnp.zeros((LANES,), jnp.float32),
                                unroll=8)
            s = jnp.sum(acc)
            return jnp.where(lanes == r, jnp.full((LANES,), s), res)

        osum[g, :] = _sqrt16(lax.fori_loop(
            0, C, row, jnp.zeros((LANES,), jnp.float32)))

        @pl.when(g + NBUF < NCHUNK)
        def _():
            start(g + NBUF, b)
        return carry

    lax.fori_loop(0, NCHUNK, step, 0)
    pltpu.sync_copy(osum, out_h.at[wid])


def kernel(positive_sample, idx, negative_sample, entity_embedding,
           relation_embedding):
    del idx, negative_sample
    ps_t = positive_sample.T.reshape(3, NW, NCHUNK, C)

    mesh = plsc.VectorSubcoreMesh(core_axis_name="c", subcore_axis_name="s")
    f = pl.kernel(
        _body,
        out_type=jax.ShapeDtypeStruct((NW, NCHUNK, C), jnp.float32),
        mesh=mesh,
        compiler_params=pltpu.CompilerParams(needs_layout_passes=False),
        scratch_types=[
            pltpu.VMEM((NCHUNK, C), jnp.int32),
            pltpu.VMEM((NCHUNK, C), jnp.int32),
            pltpu.VMEM((NCHUNK, 2 * C), jnp.int32),
            pltpu.VMEM((NCHUNK, C), jnp.int32),
            pltpu.VMEM((NBUF, 2 * C, D), jnp.float32),
            pltpu.VMEM((NBUF, C, D), jnp.float32),
            pltpu.VMEM((NCHUNK, C), jnp.float32),
            pltpu.SemaphoreType.DMA((NBUF,)),
        ],
    )
    out = f(entity_embedding, relation_embedding, ps_t)
    return out.reshape(B)


# speedup vs baseline: 1.0055x; 1.0055x over previous
import jax
import jax.numpy as jnp
from jax import lax
from jax.experimental import pallas as pl
from jax.experimental.pallas import tpu as pltpu
from jax.experimental.pallas import tpu_sc as plsc

D = 768
B = 16384
NC = 2
NS = 16
NW = NC * NS
PER_W = B // NW
C = 16
NCHUNK = PER_W // C
NBUF = 3
LANES = 16
DCH = D // LANES


def _sqrt16(x):
    i = plsc.bitcast(x, jnp.int32)
    y = plsc.bitcast((i >> 1) + 0x1FBD1DF6, jnp.float32)
    for _ in range(3):
        y = 0.5 * (y + x / y)
    return y


def _body(ee, rel_t, hidx_h, ridx_h, out_h,
          hidx, ridx, hbuf, rbuf, osum, sems):
    wid = lax.axis_index("s") * NC + lax.axis_index("c")

    pltpu.sync_copy(hidx_h.at[wid], hidx)
    pltpu.sync_copy(ridx_h.at[wid], ridx)

    def start(g, b):
        pltpu.async_copy(ee.at[hidx.at[g]], hbuf.at[b], sems.at[b])
        pltpu.async_copy(rel_t.at[ridx[g, :]], rbuf.at[b], sems.at[b])

    def wait(g, b):
        pltpu.make_async_copy(ee.at[hidx.at[g]], hbuf.at[b],
                              sems.at[b]).wait()
        pltpu.make_async_copy(rel_t.at[ridx[g, :]], rbuf.at[b],
                              sems.at[b]).wait()

    for b in range(NBUF):
        start(b, b)

    lanes = lax.iota(jnp.int32, LANES)

    def step(g, carry):
        b = lax.rem(g, NBUF)
        wait(g, b)

        def row(r, res):
            def dstep(j, acc):
                sl = pl.ds(j * LANES, LANES)
                v = hbuf[b, r, sl] + rbuf[b, r, sl] - hbuf[b, C + r, sl]
                return acc + v * v
            acc = lax.fori_loop(0, DCH, dstep,
                                jnp.zeros((LANES,), jnp.float32),
                                unroll=8)
            s = jnp.sum(acc)
            return jnp.where(lanes == r, jnp.full((LANES,), s), res)

        osum[g, :] = _sqrt16(lax.fori_loop(
            0, C, row, jnp.zeros((LANES,), jnp.float32)))

        @pl.when(g + NBUF < NCHUNK)
        def _():
            start(g + NBUF, b)
        return carry

    lax.fori_loop(0, NCHUNK, step, 0)
    pltpu.sync_copy(osum, out_h.at[wid])


def kernel(positive_sample, idx, negative_sample, entity_embedding,
           relation_embedding):
    del idx, negative_sample
    heads = positive_sample[:, 0].reshape(NW, NCHUNK, C)
    rels = positive_sample[:, 1].reshape(NW, NCHUNK, C)
    tails = positive_sample[:, 2].reshape(NW, NCHUNK, C)
    ht = jnp.concatenate([heads, tails], axis=-1)

    mesh = plsc.VectorSubcoreMesh(core_axis_name="c", subcore_axis_name="s")
    f = pl.kernel(
        _body,
        out_type=jax.ShapeDtypeStruct((NW, NCHUNK, C), jnp.float32),
        mesh=mesh,
        compiler_params=pltpu.CompilerParams(needs_layout_passes=False),
        scratch_types=[
            pltpu.VMEM((NCHUNK, 2 * C), jnp.int32),
            pltpu.VMEM((NCHUNK, C), jnp.int32),
            pltpu.VMEM((NBUF, 2 * C, D), jnp.float32),
            pltpu.VMEM((NBUF, C, D), jnp.float32),
            pltpu.VMEM((NCHUNK, C), jnp.float32),
            pltpu.SemaphoreType.DMA((NBUF,)),
        ],
    )
    out = f(entity_embedding, relation_embedding, ht, rels)
    return out.reshape(B)


# speedup vs baseline: 1.0473x; 1.0415x over previous
import jax
import jax.numpy as jnp
from jax import lax
from jax.experimental import pallas as pl
from jax.experimental.pallas import tpu as pltpu
from jax.experimental.pallas import tpu_sc as plsc

D = 768
B = 16384
NC = 2
NS = 16
NW = NC * NS
PER_W = B // NW
C = 16
NCHUNK = PER_W // C
NBUF = 3
LANES = 16
DCH = D // LANES


def _sqrt16(x):
    i = plsc.bitcast(x, jnp.int32)
    y = plsc.bitcast((i >> 1) + 0x1FBD1DF6, jnp.float32)
    for _ in range(3):
        y = 0.5 * (y + x / y)
    return y


def _body(ee, rel_t, hidx_h, ridx_h, out_h,
          hidx, ridx, hbuf, rbuf, osum, sems):
    wid = lax.axis_index("s") * NC + lax.axis_index("c")

    pltpu.sync_copy(hidx_h.at[pl.ds(wid * NCHUNK * 2 * C, NCHUNK * 2 * C)],
                    hidx)
    pltpu.sync_copy(ridx_h.at[pl.ds(wid * PER_W, PER_W)], ridx)

    def start(g, b):
        pltpu.async_copy(ee.at[hidx.at[pl.ds(g * 2 * C, 2 * C)]],
                         hbuf.at[b], sems.at[b])
        pltpu.async_copy(rel_t.at[ridx[pl.ds(g * C, C)]], rbuf.at[b],
                         sems.at[b])

    def wait(g, b):
        pltpu.make_async_copy(ee.at[hidx.at[pl.ds(g * 2 * C, 2 * C)]],
                              hbuf.at[b], sems.at[b]).wait()
        pltpu.make_async_copy(rel_t.at[ridx[pl.ds(g * C, C)]], rbuf.at[b],
                              sems.at[b]).wait()

    for b in range(NBUF):
        start(b, b)

    lanes = lax.iota(jnp.int32, LANES)

    def step(g, carry):
        b = lax.rem(g, NBUF)
        wait(g, b)

        def row(r, res):
            def dstep(j, acc):
                sl = pl.ds(j * LANES, LANES)
                v = hbuf[b, r, sl] + rbuf[b, r, sl] - hbuf[b, C + r, sl]
                return acc + v * v
            acc = lax.fori_loop(0, DCH, dstep,
                                jnp.zeros((LANES,), jnp.float32),
                                unroll=8)
            s = jnp.sum(acc)
            return jnp.where(lanes == r, jnp.full((LANES,), s), res)

        osum[pl.ds(g * C, LANES)] = _sqrt16(lax.fori_loop(
            0, C, row, jnp.zeros((LANES,), jnp.float32)))

        @pl.when(g + NBUF < NCHUNK)
        def _():
            start(g + NBUF, b)
        return carry

    lax.fori_loop(0, NCHUNK, step, 0)
    pltpu.sync_copy(osum, out_h.at[pl.ds(wid * PER_W, PER_W)])


def kernel(positive_sample, idx, negative_sample, entity_embedding,
           relation_embedding):
    del idx, negative_sample
    heads = positive_sample[:, 0].reshape(B // C, C)
    tails = positive_sample[:, 2].reshape(B // C, C)
    ht = jnp.concatenate([heads, tails], axis=-1).reshape(2 * B)
    rels = positive_sample[:, 1]

    mesh = plsc.VectorSubcoreMesh(core_axis_name="c", subcore_axis_name="s")
    f = pl.kernel(
        _body,
        out_type=jax.ShapeDtypeStruct((B,), jnp.float32),
        mesh=mesh,
        compiler_params=pltpu.CompilerParams(needs_layout_passes=False),
        scratch_types=[
            pltpu.VMEM((NCHUNK * 2 * C,), jnp.int32),
            pltpu.VMEM((PER_W,), jnp.int32),
            pltpu.VMEM((NBUF, 2 * C, D), jnp.float32),
            pltpu.VMEM((NBUF, C, D), jnp.float32),
            pltpu.VMEM((PER_W,), jnp.float32),
            pltpu.SemaphoreType.DMA((NBUF,)),
        ],
    )
    return f(entity_embedding, relation_embedding, ht, rels)
